# baseline (speedup 1.0000x reference)
import jax
import jax.numpy as jnp
from jax.experimental import pallas as pl
from jax.experimental.pallas import tpu as pltpu

_CONF_THRES = 0.5
_IOU_THRES = 0.45
_B = 1024


def _iou_tile(rx1, ry1, rx2, ry2, ar, cx1, cy1, cx2, cy2, ac):
    dx = jnp.clip(jnp.minimum(rx2, cx2) - jnp.maximum(rx1, cx1), 0.0)
    dy = jnp.clip(jnp.minimum(ry2, cy2) - jnp.maximum(ry1, cy1), 0.0)
    inter = dx * dy
    return inter / (ar + ac - inter + 1e-9)


def _nms_body(br, bt, sc, keep_c, area_c):
    np_ = keep_c.shape[1]
    nb = np_ // _B

    valid = jnp.where(sc[...] > _CONF_THRES, 1.0, 0.0)
    keep_c[...] = valid
    area_c[...] = (bt[2:3, :] - bt[0:1, :]) * (bt[3:4, :] - bt[1:2, :])

    nv = jnp.sum(valid).astype(jnp.int32)
    nb_eff = jnp.minimum((nv + _B - 1) // _B, nb)

    eye = (
        jax.lax.broadcasted_iota(jnp.int32, (_B, _B), 0)
        == jax.lax.broadcasted_iota(jnp.int32, (_B, _B), 1)
    ).astype(jnp.float32)
    tri_u = (
        jax.lax.broadcasted_iota(jnp.int32, (_B, _B), 0)
        < jax.lax.broadcasted_iota(jnp.int32, (_B, _B), 1)
    )

    def transpose_vec(kc):
        return jnp.sum(eye * kc, axis=1, keepdims=True)

    def block_step(k, _):
        kb = k * _B
        bx1r = br[pl.ds(kb, _B), 0:1]
        by1r = br[pl.ds(kb, _B), 1:2]
        bx2r = br[pl.ds(kb, _B), 2:3]
        by2r = br[pl.ds(kb, _B), 3:4]
        bx1c = bt[0:1, pl.ds(kb, _B)]
        by1c = bt[1:2, pl.ds(kb, _B)]
        bx2c = bt[2:3, pl.ds(kb, _B)]
        by2c = bt[3:4, pl.ds(kb, _B)]
        bac = (bx2c - bx1c) * (by2c - by1c)
        bar = (bx2r - bx1r) * (by2r - by1r)

        iou_kk = _iou_tile(
            bx1r, by1r, bx2r, by2r, bar, bx1c, by1c, bx2c, by2c, bac
        )
        ovu = jnp.where((iou_kk > _IOU_THRES) & tri_u, 1.0, 0.0)

        valb_c = keep_c[:, pl.ds(kb, _B)]

        def cond(carry):
            return carry[2] > 0

        def body(carry):
            kr, kc, _ = carry
            sup_c = jnp.max(ovu * kr, axis=0, keepdims=True)
            kc_new = valb_c * (1.0 - jnp.where(sup_c > 0.0, 1.0, 0.0))
            changed = (jnp.sum(jnp.abs(kc_new - kc)) > 0.0).astype(jnp.int32)
            return transpose_vec(kc_new), kc_new, changed

        kr_f, kc_f, _ = jax.lax.while_loop(
            cond, body, (transpose_vec(valb_c), valb_c, jnp.int32(1))
        )
        keep_c[:, pl.ds(kb, _B)] = kc_f

        def tail_step(j, _):
            jb = j * _B
            cx1 = bt[0:1, pl.ds(jb, _B)]
            cy1 = bt[1:2, pl.ds(jb, _B)]
            cx2 = bt[2:3, pl.ds(jb, _B)]
            cy2 = bt[3:4, pl.ds(jb, _B)]
            cac = area_c[:, pl.ds(jb, _B)]

            iou_kj = _iou_tile(
                bx1r, by1r, bx2r, by2r, bar, cx1, cy1, cx2, cy2, cac
            )
            sup_c = jnp.max(iou_kj * kr_f, axis=0, keepdims=True) > _IOU_THRES
            keep_c[:, pl.ds(jb, _B)] *= 1.0 - sup_c.astype(jnp.float32)
            return 0

        jax.lax.fori_loop(k + 1, nb_eff, tail_step, 0)
        return 0

    jax.lax.fori_loop(0, nb_eff, block_step, 0)


def kernel(boxes, scores):
    n = scores.shape[0]
    np_ = ((n + _B - 1) // _B) * _B

    neg_s, order = jax.lax.sort_key_val(-scores, jnp.arange(n, dtype=jnp.int32))
    boxes_s = boxes[order]

    boxes_p = jnp.zeros((np_, 4), jnp.float32).at[:n].set(boxes_s)
    scores_p = jnp.zeros((1, np_), jnp.float32).at[0, :n].set(-neg_s)

    keep_s = pl.pallas_call(
        _nms_body,
        out_shape=jax.ShapeDtypeStruct((1, np_), jnp.float32),
        scratch_shapes=[pltpu.VMEM((1, np_), jnp.float32)],
    )(boxes_p, boxes_p.T, scores_p)[0, :n]

    kf = jnp.zeros((n,), jnp.float32).at[order].set(keep_s)
    det5 = jnp.concatenate([boxes, scores[:, None]], axis=1)
    return det5 * kf[:, None]


# speedup vs baseline: 1.0774x; 1.0774x over previous
import jax
import jax.numpy as jnp
from jax.experimental import pallas as pl
from jax.experimental.pallas import tpu as pltpu

_CONF_THRES = 0.5
_IOU_THRES = 0.45
_B = 512


def _iou_tile(rx1, ry1, rx2, ry2, ar, cx1, cy1, cx2, cy2, ac):
    dx = jnp.clip(jnp.minimum(rx2, cx2) - jnp.maximum(rx1, cx1), 0.0)
    dy = jnp.clip(jnp.minimum(ry2, cy2) - jnp.maximum(ry1, cy1), 0.0)
    inter = dx * dy
    return inter / (ar + ac - inter + 1e-9)


def _nms_body(br, bt, sc, keep_c, area_c):
    np_ = keep_c.shape[1]
    nb = np_ // _B

    valid = jnp.where(sc[...] > _CONF_THRES, 1.0, 0.0)
    keep_c[...] = valid
    area_c[...] = (bt[2:3, :] - bt[0:1, :]) * (bt[3:4, :] - bt[1:2, :])

    nv = jnp.sum(valid).astype(jnp.int32)
    nb_eff = jnp.minimum((nv + _B - 1) // _B, nb)

    tri_u = (
        jax.lax.broadcasted_iota(jnp.int32, (_B, _B), 0)
        < jax.lax.broadcasted_iota(jnp.int32, (_B, _B), 1)
    )

    def suppress_counts(kc, ovm):
        return jax.lax.dot_general(
            kc, ovm, (((1,), (0,)), ((), ())),
            preferred_element_type=jnp.float32,
        )

    def block_step(k, _):
        kb = k * _B
        bx1r = br[pl.ds(kb, _B), 0:1]
        by1r = br[pl.ds(kb, _B), 1:2]
        bx2r = br[pl.ds(kb, _B), 2:3]
        by2r = br[pl.ds(kb, _B), 3:4]
        bx1c = bt[0:1, pl.ds(kb, _B)]
        by1c = bt[1:2, pl.ds(kb, _B)]
        bx2c = bt[2:3, pl.ds(kb, _B)]
        by2c = bt[3:4, pl.ds(kb, _B)]
        bac = (bx2c - bx1c) * (by2c - by1c)
        bar = (bx2r - bx1r) * (by2r - by1r)

        iou_kk = _iou_tile(
            bx1r, by1r, bx2r, by2r, bar, bx1c, by1c, bx2c, by2c, bac
        )
        ovu = jnp.where((iou_kk > _IOU_THRES) & tri_u, 1.0, 0.0)

        valb_c = keep_c[:, pl.ds(kb, _B)]

        def cond(carry):
            return carry[1] > 0

        def body(carry):
            kc, _ = carry
            sup_c = suppress_counts(kc, ovu)
            kc_new = valb_c * (1.0 - jnp.where(sup_c > 0.0, 1.0, 0.0))
            changed = (jnp.sum(jnp.abs(kc_new - kc)) > 0.0).astype(jnp.int32)
            return kc_new, changed

        kc_f, _ = jax.lax.while_loop(cond, body, (valb_c, jnp.int32(1)))
        keep_c[:, pl.ds(kb, _B)] = kc_f

        def tail_step(j, _):
            jb = j * _B
            cx1 = bt[0:1, pl.ds(jb, _B)]
            cy1 = bt[1:2, pl.ds(jb, _B)]
            cx2 = bt[2:3, pl.ds(jb, _B)]
            cy2 = bt[3:4, pl.ds(jb, _B)]
            cac = area_c[:, pl.ds(jb, _B)]

            iou_kj = _iou_tile(
                bx1r, by1r, bx2r, by2r, bar, cx1, cy1, cx2, cy2, cac
            )
            ovm = jnp.where(iou_kj > _IOU_THRES, 1.0, 0.0)
            sup_c = suppress_counts(kc_f, ovm) > 0.0
            keep_c[:, pl.ds(jb, _B)] *= 1.0 - sup_c.astype(jnp.float32)
            return 0

        jax.lax.fori_loop(k + 1, nb_eff, tail_step, 0)
        return 0

    jax.lax.fori_loop(0, nb_eff, block_step, 0)


def kernel(boxes, scores):
    n = scores.shape[0]
    np_ = ((n + _B - 1) // _B) * _B

    neg_s, order = jax.lax.sort_key_val(-scores, jnp.arange(n, dtype=jnp.int32))
    boxes_s = boxes[order]

    boxes_p = jnp.zeros((np_, 4), jnp.float32).at[:n].set(boxes_s)
    scores_p = jnp.zeros((1, np_), jnp.float32).at[0, :n].set(-neg_s)

    keep_s = pl.pallas_call(
        _nms_body,
        out_shape=jax.ShapeDtypeStruct((1, np_), jnp.float32),
        scratch_shapes=[pltpu.VMEM((1, np_), jnp.float32)],
    )(boxes_p, boxes_p.T, scores_p)[0, :n]

    kf = jnp.zeros((n,), jnp.float32).at[order].set(keep_s)
    det5 = jnp.concatenate([boxes, scores[:, None]], axis=1)
    return det5 * kf[:, None]


# speedup vs baseline: 1.0794x; 1.0019x over previous
import jax
import jax.numpy as jnp
from jax import lax
from jax.experimental import pallas as pl
from jax.experimental.pallas import tpu as pltpu
from jax.experimental.pallas import tpu_sc as plsc

_CONF_THRES = 0.5
_IOU_THRES = 0.45
_B = 512


def _iou_tile(rx1, ry1, rx2, ry2, ar, cx1, cy1, cx2, cy2, ac):
    dx = jnp.clip(jnp.minimum(rx2, cx2) - jnp.maximum(rx1, cx1), 0.0)
    dy = jnp.clip(jnp.minimum(ry2, cy2) - jnp.maximum(ry1, cy1), 0.0)
    inter = dx * dy
    return inter / (ar + ac - inter + 1e-9)


def _nms_body(br, bt, sc, keep_c, area_c):
    np_ = keep_c.shape[1]
    nb = np_ // _B

    valid = jnp.where(sc[...] > _CONF_THRES, 1.0, 0.0)
    keep_c[...] = valid
    area_c[...] = (bt[2:3, :] - bt[0:1, :]) * (bt[3:4, :] - bt[1:2, :])

    nv = jnp.sum(valid).astype(jnp.int32)
    nb_eff = jnp.minimum((nv + _B - 1) // _B, nb)

    tri_u = (
        jax.lax.broadcasted_iota(jnp.int32, (_B, _B), 0)
        < jax.lax.broadcasted_iota(jnp.int32, (_B, _B), 1)
    )

    def suppress_counts(kc, ovm):
        return jax.lax.dot_general(
            kc, ovm, (((1,), (0,)), ((), ())),
            preferred_element_type=jnp.float32,
        )

    def block_step(k, _):
        kb = k * _B
        bx1r = br[pl.ds(kb, _B), 0:1]
        by1r = br[pl.ds(kb, _B), 1:2]
        bx2r = br[pl.ds(kb, _B), 2:3]
        by2r = br[pl.ds(kb, _B), 3:4]
        bx1c = bt[0:1, pl.ds(kb, _B)]
        by1c = bt[1:2, pl.ds(kb, _B)]
        bx2c = bt[2:3, pl.ds(kb, _B)]
        by2c = bt[3:4, pl.ds(kb, _B)]
        bac = (bx2c - bx1c) * (by2c - by1c)
        bar = (bx2r - bx1r) * (by2r - by1r)

        iou_kk = _iou_tile(
            bx1r, by1r, bx2r, by2r, bar, bx1c, by1c, bx2c, by2c, bac
        )
        ovu = jnp.where((iou_kk > _IOU_THRES) & tri_u, 1.0, 0.0)

        valb_c = keep_c[:, pl.ds(kb, _B)]

        def cond(carry):
            return carry[1] > 0

        def body(carry):
            kc, _ = carry
            sup_c = suppress_counts(kc, ovu)
            kc_new = valb_c * (1.0 - jnp.where(sup_c > 0.0, 1.0, 0.0))
            changed = (jnp.sum(jnp.abs(kc_new - kc)) > 0.0).astype(jnp.int32)
            return kc_new, changed

        kc_f, _ = jax.lax.while_loop(cond, body, (valb_c, jnp.int32(1)))
        keep_c[:, pl.ds(kb, _B)] = kc_f

        def tail_step(j, _):
            jb = j * _B
            cx1 = bt[0:1, pl.ds(jb, _B)]
            cy1 = bt[1:2, pl.ds(jb, _B)]
            cx2 = bt[2:3, pl.ds(jb, _B)]
            cy2 = bt[3:4, pl.ds(jb, _B)]
            cac = area_c[:, pl.ds(jb, _B)]

            iou_kj = _iou_tile(
                bx1r, by1r, bx2r, by2r, bar, cx1, cy1, cx2, cy2, cac
            )
            ovm = jnp.where(iou_kj > _IOU_THRES, 1.0, 0.0)
            sup_c = suppress_counts(kc_f, ovm) > 0.0
            keep_c[:, pl.ds(jb, _B)] *= 1.0 - sup_c.astype(jnp.float32)
            return 0

        jax.lax.fori_loop(k + 1, nb_eff, tail_step, 0)
        return 0

    jax.lax.fori_loop(0, nb_eff, block_step, 0)


def _make_sc_gather(n, np_):
    r = 256
    nw = np_ // r
    assert np_ % r == 0 and nw <= 32

    def body(boxes_hbm, order_hbm, br_hbm, bt_hbm, tbl_v, idx_v, br_v, bt_v):
        wid = lax.axis_index("s") * 2 + lax.axis_index("c")

        @pl.when(wid < nw)
        def _():
            base = wid * r
            pltpu.sync_copy(boxes_hbm, tbl_v)
            pltpu.sync_copy(order_hbm.at[pl.ds(base, r)], idx_v)
            for i in range(r // 16):
                idx16 = idx_v[pl.ds(i * 16, 16)]
                flat_row = (lax.iota(jnp.int32, 16) + (i * 16)) * 4
                for c in range(4):
                    vals = plsc.load_gather(tbl_v, [idx16 * 4 + c])
                    bt_v[c, pl.ds(i * 16, 16)] = vals
                    plsc.store_scatter(br_v, [flat_row + c], vals)
            pltpu.sync_copy(br_v, br_hbm.at[pl.ds(base * 4, r * 4)])
            pltpu.sync_copy(bt_v, bt_hbm.at[:, pl.ds(base, r)])

    return pl.kernel(
        body,
        out_type=(
            jax.ShapeDtypeStruct((np_ * 4,), jnp.float32),
            jax.ShapeDtypeStruct((4, np_), jnp.float32),
        ),
        mesh=plsc.VectorSubcoreMesh(core_axis_name="c", subcore_axis_name="s"),
        compiler_params=pltpu.CompilerParams(needs_layout_passes=False),
        scratch_types=[
            pltpu.VMEM((n * 4,), jnp.float32),
            pltpu.VMEM((r,), jnp.int32),
            pltpu.VMEM((r * 4,), jnp.float32),
            pltpu.VMEM((4, r), jnp.float32),
        ],
    )


def kernel(boxes, scores):
    n = scores.shape[0]
    np_ = ((n + _B - 1) // _B) * _B

    neg_s, order = jax.lax.sort_key_val(-scores, jnp.arange(n, dtype=jnp.int32))

    order_p = jnp.concatenate(
        [order, jnp.zeros((np_ - n,), jnp.int32)]
    )
    boxes_pf, boxes_t = _make_sc_gather(n, np_)(boxes.reshape(-1), order_p)
    boxes_p = boxes_pf.reshape(np_, 4)
    scores_p = jnp.zeros((1, np_), jnp.float32).at[0, :n].set(-neg_s)

    keep_s = pl.pallas_call(
        _nms_body,
        out_shape=jax.ShapeDtypeStruct((1, np_), jnp.float32),
        scratch_shapes=[pltpu.VMEM((1, np_), jnp.float32)],
    )(boxes_p, boxes_t, scores_p)[0, :n]

    kf = jnp.zeros((n,), jnp.float32).at[order].set(keep_s)
    det5 = jnp.concatenate([boxes, scores[:, None]], axis=1)
    return det5 * kf[:, None]
